# Initial kernel scaffold; baseline (speedup 1.0000x reference)
import jax
import jax.numpy as jnp
from jax.experimental import pallas as pl


def kernel(pred_logits, pred_boxes, target_sizes, select_box_nums_for_evaluation):
    raise NotImplementedError("write your pallas kernel here")


# speedup vs baseline: 38.1721x; 38.1721x over previous
import jax
import jax.numpy as jnp
from jax import lax
from jax.experimental import pallas as pl
from jax.experimental.pallas import tpu as pltpu

N_TOPK = 10000
BK = 256
NPAD = 10240
M_BLOCKS = NPAD // BK
K_OUT = 300
IOU_THR = 0.7


def _nms_body(x1r_, y1r_, x2r_, y2r_, arr_, keep_out_, colbuf, cnt):
    x1r, y1r, x2r, y2r, arr, keep_out = (
        x1r_.at[0], y1r_.at[0], x2r_.at[0], y2r_.at[0], arr_.at[0],
        keep_out_.at[0])
    keep_out[...] = jnp.zeros((1, NPAD), jnp.float32)
    cnt[0] = 0

    col_i = lax.broadcasted_iota(jnp.int32, (1, BK), 1)
    eye = (lax.broadcasted_iota(jnp.int32, (BK, BK), 0)
           == lax.broadcasted_iota(jnp.int32, (BK, BK), 1))

    def block_step(bi, carry):
        @pl.when(cnt[0] < K_OUT)
        def _():
            base = pl.multiple_of(bi * BK, BK)
            x1 = x1r[0:1, pl.ds(base, BK)]
            y1 = y1r[0:1, pl.ds(base, BK)]
            x2 = x2r[0:1, pl.ds(base, BK)]
            y2 = y2r[0:1, pl.ds(base, BK)]
            ar = arr[0:1, pl.ds(base, BK)]

            def run_pulls(sup_ref):
                sup_ref[...] = jnp.zeros((1, BK), jnp.float32)

                def pull(bj, carry):
                    @pl.when(bj < bi)
                    def _inner():
                        pb = pl.multiple_of(bj * BK, BK)
                        px1 = colbuf[pl.ds(pb, BK), 0:1]
                        py1 = colbuf[pl.ds(pb, BK), 1:2]
                        px2 = colbuf[pl.ds(pb, BK), 2:3]
                        py2 = colbuf[pl.ds(pb, BK), 3:4]
                        par = colbuf[pl.ds(pb, BK), 4:5]
                        pk = colbuf[pl.ds(pb, BK), 5:6]
                        w = jnp.maximum(
                            jnp.minimum(px2, x2) - jnp.maximum(px1, x1), 0.0)
                        h = jnp.maximum(
                            jnp.minimum(py2, y2) - jnp.maximum(py1, y1), 0.0)
                        inter = w * h
                        iou = inter / (par + ar - inter + 1e-9)
                        hit = jnp.where((iou > IOU_THR) & (pk > 0.5), 1.0, 0.0)
                        sup_ref[...] = jnp.maximum(
                            sup_ref[...], jnp.max(hit, axis=0, keepdims=True))
                    return carry

                lax.fori_loop(0, M_BLOCKS, pull, 0)
                sup = sup_ref[...]
                gidx = base + col_i
                kf = jnp.where((sup < 0.5) & (gidx < N_TOPK), 1.0, 0.0)

                def intra(i, kf):
                    onehot = col_i == i
                    sel = lambda v: jnp.sum(jnp.where(onehot, v, 0.0))
                    xi1, yi1 = sel(x1), sel(y1)
                    xi2, yi2 = sel(x2), sel(y2)
                    ai = sel(ar)
                    ki = jnp.sum(jnp.where(onehot, kf, 0.0))
                    w = jnp.maximum(jnp.minimum(xi2, x2)
                                    - jnp.maximum(xi1, x1), 0.0)
                    h = jnp.maximum(jnp.minimum(yi2, y2)
                                    - jnp.maximum(yi1, y1), 0.0)
                    inter = w * h
                    iou = inter / (ai + ar - inter + 1e-9)
                    sup_i = (iou > IOU_THR) & (col_i > i) & (ki > 0.5)
                    return jnp.where(sup_i, 0.0, kf)

                kf = lax.fori_loop(0, BK, intra, kf)
                keep_out[0:1, pl.ds(base, BK)] = kf

                tocol = lambda v: jnp.sum(
                    jnp.where(eye, v, 0.0), axis=1, keepdims=True)
                colbuf[pl.ds(base, BK), 0:1] = tocol(x1)
                colbuf[pl.ds(base, BK), 1:2] = tocol(y1)
                colbuf[pl.ds(base, BK), 2:3] = tocol(x2)
                colbuf[pl.ds(base, BK), 3:4] = tocol(y2)
                colbuf[pl.ds(base, BK), 4:5] = tocol(ar)
                colbuf[pl.ds(base, BK), 5:6] = tocol(kf)
                cnt[0] = cnt[0] + jnp.sum(kf).astype(jnp.int32)

            pl.run_scoped(run_pulls, pltpu.VMEM((1, BK), jnp.float32))
        return carry

    lax.fori_loop(0, M_BLOCKS, block_step, 0)


def _nms_keep(x1, y1, x2, y2, ar):
    bs = x1.shape[0]
    spec = pl.BlockSpec((1, 1, NPAD), lambda b: (b, 0, 0))
    r3 = lambda v: v.reshape(bs, 1, NPAD)
    out = pl.pallas_call(
        _nms_body,
        grid=(bs,),
        in_specs=[spec] * 5,
        out_specs=spec,
        out_shape=jax.ShapeDtypeStruct((bs, 1, NPAD), jnp.float32),
        scratch_shapes=[
            pltpu.VMEM((NPAD, 6), jnp.float32),
            pltpu.SMEM((1,), jnp.int32),
        ],
    )(r3(x1), r3(y1), r3(x2), r3(y2), r3(ar))
    return out.reshape(bs, NPAD)


def kernel(pred_logits, pred_boxes, target_sizes,
           select_box_nums_for_evaluation):
    bs, nq, nc = pred_logits.shape
    n = N_TOPK

    all_scores = jax.nn.sigmoid(pred_logits).reshape(bs, nq * nc)
    cx, cy, w, h = (pred_boxes[..., 0], pred_boxes[..., 1],
                    pred_boxes[..., 2], pred_boxes[..., 3])
    boxes_xyxy = jnp.stack(
        [cx - 0.5 * w, cy - 0.5 * h, cx + 0.5 * w, cy + 0.5 * h], axis=-1)
    img_h = target_sizes[:, 0].astype(jnp.float32)
    img_w = target_sizes[:, 1].astype(jnp.float32)
    scale_fct = jnp.stack([img_w, img_h, img_w, img_h], axis=1)
    boxes_scaled = boxes_xyxy * scale_fct[:, None, :]

    sc, topi = lax.top_k(all_scores, n)
    bidx = topi // nc
    lbl = topi % nc
    box = jnp.take_along_axis(
        boxes_scaled, jnp.broadcast_to(bidx[:, :, None], (bs, n, 4)), axis=1)

    max_coord = jnp.max(box, axis=(1, 2)) + 1.0
    offs = lbl.astype(jnp.float32) * max_coord[:, None]
    boff = box + offs[:, :, None]
    x1, y1, x2, y2 = (boff[..., 0], boff[..., 1], boff[..., 2], boff[..., 3])
    areas = (x2 - x1) * (y2 - y1)

    pad = ((0, 0), (0, NPAD - n))
    keep = _nms_keep(jnp.pad(x1, pad), jnp.pad(y1, pad),
                     jnp.pad(x2, pad), jnp.pad(y2, pad), jnp.pad(areas, pad))
    keepb = keep[:, :n] > 0.5

    idx = jnp.arange(n, dtype=jnp.int32)
    ranks = jnp.sort(jnp.where(keepb, idx[None, :], n), axis=1)[:, :K_OUT]
    valid = ranks < n
    inds = jnp.clip(ranks, 0, n - 1)
    out_boxes = jnp.take_along_axis(
        box, jnp.broadcast_to(inds[:, :, None], (bs, K_OUT, 4)), axis=1)
    out_scores = jnp.take_along_axis(sc, inds, axis=1)
    out_labels = jnp.take_along_axis(lbl, inds, axis=1)
    valid = valid & (jnp.arange(K_OUT)[None, :]
                     < select_box_nums_for_evaluation)
    return out_boxes, out_scores, out_labels, inds, valid


# speedup vs baseline: 96.1601x; 2.5191x over previous
import jax
import jax.numpy as jnp
from jax import lax
from jax.experimental import pallas as pl
from jax.experimental.pallas import tpu as pltpu

N_TOPK = 10000
BK = 256
NPAD = 10240
K_OUT = 300
IOU_THR = 0.7
P_PRE = 1024


def _nms_body(x1r_, y1r_, x2r_, y2r_, arr_, keep_out_, colbuf, cnt):
    x1r, y1r, x2r, y2r, arr, keep_out = (
        x1r_.at[0], y1r_.at[0], x2r_.at[0], y2r_.at[0], arr_.at[0],
        keep_out_.at[0])
    w_tot = x1r_.shape[2]
    m_blocks = w_tot // BK
    keep_out[...] = jnp.zeros((1, w_tot), jnp.float32)
    cnt[0] = 0
    cnt[1] = 0

    col_i = lax.broadcasted_iota(jnp.int32, (1, BK), 1)
    row_m = lax.broadcasted_iota(jnp.int32, (BK, BK), 0)
    col_m = lax.broadcasted_iota(jnp.int32, (BK, BK), 1)
    eye = row_m == col_m
    tri = row_m < col_m

    def block_step(bi, carry):
        @pl.when(cnt[0] < K_OUT)
        def _():
            base = pl.multiple_of(bi * BK, BK)
            x1 = x1r[0:1, pl.ds(base, BK)]
            y1 = y1r[0:1, pl.ds(base, BK)]
            x2 = x2r[0:1, pl.ds(base, BK)]
            y2 = y2r[0:1, pl.ds(base, BK)]
            ar = arr[0:1, pl.ds(base, BK)]

            def run_pulls(sup_ref):
                sup_ref[...] = jnp.zeros((1, BK), jnp.float32)

                def pull(bj, carry2):
                    @pl.when(bj < bi)
                    def _inner():
                        pb = pl.multiple_of(bj * BK, BK)
                        px1 = colbuf[pl.ds(pb, BK), 0:1]
                        py1 = colbuf[pl.ds(pb, BK), 1:2]
                        px2 = colbuf[pl.ds(pb, BK), 2:3]
                        py2 = colbuf[pl.ds(pb, BK), 3:4]
                        par = colbuf[pl.ds(pb, BK), 4:5]
                        pk = colbuf[pl.ds(pb, BK), 5:6]
                        w = jnp.maximum(
                            jnp.minimum(px2, x2) - jnp.maximum(px1, x1), 0.0)
                        h = jnp.maximum(
                            jnp.minimum(py2, y2) - jnp.maximum(py1, y1), 0.0)
                        inter = w * h
                        iou = inter / (par + ar - inter + 1e-9)
                        hit = jnp.where((iou > IOU_THR) & (pk > 0.5), 1.0, 0.0)
                        sup_ref[...] = jnp.maximum(
                            sup_ref[...], jnp.max(hit, axis=0, keepdims=True))
                    return carry2

                lax.fori_loop(0, m_blocks, pull, 0)
                sup = sup_ref[...]
                gidx = base + col_i
                init = jnp.where((sup < 0.5) & (gidx < N_TOPK), 1.0, 0.0)

                tocol = lambda v: jnp.sum(
                    jnp.where(eye, v, 0.0), axis=1, keepdims=True)
                cx1, cy1 = tocol(x1), tocol(y1)
                cx2, cy2 = tocol(x2), tocol(y2)
                car = tocol(ar)
                colbuf[pl.ds(base, BK), 0:1] = cx1
                colbuf[pl.ds(base, BK), 1:2] = cy1
                colbuf[pl.ds(base, BK), 2:3] = cx2
                colbuf[pl.ds(base, BK), 3:4] = cy2
                colbuf[pl.ds(base, BK), 4:5] = car

                w_m = jnp.maximum(
                    jnp.minimum(cx2, x2) - jnp.maximum(cx1, x1), 0.0)
                h_m = jnp.maximum(
                    jnp.minimum(cy2, y2) - jnp.maximum(cy1, y1), 0.0)
                inter = w_m * h_m
                iou = inter / (car + ar - inter + 1e-9)
                amat = (iou > IOU_THR) & tri

                sup_ref[...] = init
                cnt[1] = 0

                def fix(t, c):
                    @pl.when(cnt[1] == 0)
                    def _inner():
                        k = sup_ref[...]
                        kcol = jnp.sum(
                            jnp.where(eye, k, 0.0), axis=1, keepdims=True)
                        s = jnp.max(
                            jnp.where(amat & (kcol > 0.5), 1.0, 0.0),
                            axis=0, keepdims=True)
                        knew = jnp.where(s < 0.5, init, 0.0)
                        sup_ref[...] = knew
                        cnt[1] = jnp.where(
                            jnp.sum(jnp.abs(knew - k)) == 0.0, 1, 0)
                    return c

                lax.fori_loop(0, BK, fix, 0)
                kf = sup_ref[...]
                keep_out[0:1, pl.ds(base, BK)] = kf
                colbuf[pl.ds(base, BK), 5:6] = tocol(kf)
                cnt[0] = cnt[0] + jnp.sum(kf).astype(jnp.int32)

            pl.run_scoped(run_pulls, pltpu.VMEM((1, BK), jnp.float32))
        return carry

    lax.fori_loop(0, m_blocks, block_step, 0)


def _nms_keep(x1, y1, x2, y2, ar):
    bs, w_tot = x1.shape
    spec = pl.BlockSpec((1, 1, w_tot), lambda b: (b, 0, 0))
    r3 = lambda v: v.reshape(bs, 1, w_tot)
    out = pl.pallas_call(
        _nms_body,
        grid=(bs,),
        in_specs=[spec] * 5,
        out_specs=spec,
        out_shape=jax.ShapeDtypeStruct((bs, 1, w_tot), jnp.float32),
        scratch_shapes=[
            pltpu.VMEM((w_tot, 6), jnp.float32),
            pltpu.SMEM((2,), jnp.int32),
        ],
    )(r3(x1), r3(y1), r3(x2), r3(y2), r3(ar))
    return out.reshape(bs, w_tot)


def _thresh_body(keys_ref, qmax_ref, mc_ref):
    keys = keys_ref[0]
    nq, nc = keys.shape
    qmax = qmax_ref[0]

    def step_t(it, lohi):
        lo, hi = lohi
        mid = (lo + hi) // 2
        cnt = jnp.sum(jnp.where(keys >= mid, 1, 0))
        big = cnt >= N_TOPK
        return (jnp.where(big, mid, lo), jnp.where(big, hi, mid))

    lo, hi = lax.fori_loop(
        0, 31, step_t, (jnp.int32(0), jnp.int32(0x7F800000)))
    t_val = lo
    cnt_gt = jnp.sum(jnp.where(keys > t_val, 1, 0))
    m_ties = N_TOPK - cnt_gt

    idx2d = (lax.broadcasted_iota(jnp.int32, (nq, nc), 0) * nc
             + lax.broadcasted_iota(jnp.int32, (nq, nc), 1))
    ties = keys == t_val

    def step_x(it, lohi):
        lo, hi = lohi
        mid = (lo + hi) // 2
        cnt = jnp.sum(jnp.where(ties & (idx2d < mid), 1, 0))
        big = cnt >= m_ties
        return (jnp.where(big, lo, mid), jnp.where(big, mid, hi))

    xlo, xhi = lax.fori_loop(
        0, 18, step_x, (jnp.int32(0), jnp.int32(nq * nc + 1)))
    x_cut = xhi

    selected = (keys > t_val) | (ties & (idx2d < x_cut))
    qsel = jnp.max(jnp.where(selected, 1.0, 0.0), axis=1, keepdims=True)
    mc = jnp.max(jnp.where(qsel > 0.5, qmax, -jnp.inf)) + 1.0
    mc_ref[0] = jnp.broadcast_to(mc, (1, 1))


def _max_coord(keys, qmax):
    bs, nq, nc = keys.shape
    out = pl.pallas_call(
        _thresh_body,
        grid=(bs,),
        in_specs=[pl.BlockSpec((1, nq, nc), lambda b: (b, 0, 0)),
                  pl.BlockSpec((1, nq, 1), lambda b: (b, 0, 0))],
        out_specs=pl.BlockSpec((1, 1, 1), lambda b: (b, 0, 0)),
        out_shape=jax.ShapeDtypeStruct((bs, 1, 1), jnp.float32),
    )(keys, qmax)
    return out.reshape(bs)


def _gather_cands(boxes_scaled, sc, topi, nc, mc):
    bs = sc.shape[0]
    n = sc.shape[1]
    bidx = topi // nc
    lbl = topi % nc
    box = jnp.take_along_axis(
        boxes_scaled, jnp.broadcast_to(bidx[:, :, None], (bs, n, 4)), axis=1)
    offs = lbl.astype(jnp.float32) * mc[:, None]
    boff = box + offs[:, :, None]
    x1, y1, x2, y2 = (boff[..., 0], boff[..., 1], boff[..., 2], boff[..., 3])
    areas = (x2 - x1) * (y2 - y1)
    return box, lbl, x1, y1, x2, y2, areas


def _assemble(box, sc, lbl, keepb, select, n_virtual):
    bs, w_tot = keepb.shape
    idx = jnp.arange(w_tot, dtype=jnp.int32)
    ranks = jnp.sort(
        jnp.where(keepb, idx[None, :], n_virtual), axis=1)[:, :K_OUT]
    valid = ranks < n_virtual
    inds = jnp.clip(ranks, 0, w_tot - 1)
    out_boxes = jnp.take_along_axis(
        box, jnp.broadcast_to(inds[:, :, None], (bs, K_OUT, 4)), axis=1)
    out_scores = jnp.take_along_axis(sc, inds, axis=1)
    out_labels = jnp.take_along_axis(lbl, inds, axis=1)
    valid = valid & (jnp.arange(K_OUT)[None, :] < select)
    return out_boxes, out_scores, out_labels, inds, valid


def kernel(pred_logits, pred_boxes, target_sizes,
           select_box_nums_for_evaluation):
    bs, nq, nc = pred_logits.shape
    n = N_TOPK

    scores3 = jax.nn.sigmoid(pred_logits)
    all_scores = scores3.reshape(bs, nq * nc)
    cx, cy, w, h = (pred_boxes[..., 0], pred_boxes[..., 1],
                    pred_boxes[..., 2], pred_boxes[..., 3])
    boxes_xyxy = jnp.stack(
        [cx - 0.5 * w, cy - 0.5 * h, cx + 0.5 * w, cy + 0.5 * h], axis=-1)
    img_h = target_sizes[:, 0].astype(jnp.float32)
    img_w = target_sizes[:, 1].astype(jnp.float32)
    scale_fct = jnp.stack([img_w, img_h, img_w, img_h], axis=1)
    boxes_scaled = boxes_xyxy * scale_fct[:, None, :]

    keys = lax.bitcast_convert_type(scores3, jnp.int32)
    qmax = jnp.max(boxes_scaled, axis=2, keepdims=True)
    mc = _max_coord(keys, qmax)

    scp, topip = lax.top_k(all_scores, P_PRE)
    boxp, lblp, px1, py1, px2, py2, par = _gather_cands(
        boxes_scaled, scp, topip, nc, mc)
    keep_p = _nms_keep(px1, py1, px2, py2, par)
    cnt_p = jnp.sum(keep_p, axis=1)
    prefix_ok = jnp.all(cnt_p >= K_OUT)

    def prefix_path(_):
        return _assemble(boxp, scp, lblp, keep_p > 0.5,
                         select_box_nums_for_evaluation, n)

    def full_path(_):
        sc, topi = lax.top_k(all_scores, n)
        box, lbl, x1, y1, x2, y2, areas = _gather_cands(
            boxes_scaled, sc, topi, nc, mc)
        pad = ((0, 0), (0, NPAD - n))
        keep = _nms_keep(jnp.pad(x1, pad), jnp.pad(y1, pad),
                         jnp.pad(x2, pad), jnp.pad(y2, pad),
                         jnp.pad(areas, pad))
        return _assemble(box, sc, lbl, keep[:, :n] > 0.5,
                         select_box_nums_for_evaluation, n)

    return lax.cond(prefix_ok, prefix_path, full_path, 0)


# speedup vs baseline: 151.0546x; 1.5709x over previous
import jax
import jax.numpy as jnp
from jax import lax
from jax.experimental import pallas as pl
from jax.experimental.pallas import tpu as pltpu

N_TOPK = 10000
BK = 256
NPAD = 10240
K_OUT = 300
IOU_THR = 0.7
P_PRE = 512


def _nms_body(x1r_, y1r_, x2r_, y2r_, arr_, keep_out_, colbuf, cnt):
    x1r, y1r, x2r, y2r, arr, keep_out = (
        x1r_.at[0], y1r_.at[0], x2r_.at[0], y2r_.at[0], arr_.at[0],
        keep_out_.at[0])
    w_tot = x1r_.shape[2]
    m_blocks = w_tot // BK
    keep_out[...] = jnp.zeros((1, w_tot), jnp.float32)
    cnt[0] = 0
    cnt[1] = 0

    col_i = lax.broadcasted_iota(jnp.int32, (1, BK), 1)
    row_m = lax.broadcasted_iota(jnp.int32, (BK, BK), 0)
    col_m = lax.broadcasted_iota(jnp.int32, (BK, BK), 1)
    eye = row_m == col_m
    tri = row_m < col_m

    def block_step(bi, carry):
        @pl.when(cnt[0] < K_OUT)
        def _():
            base = pl.multiple_of(bi * BK, BK)
            x1 = x1r[0:1, pl.ds(base, BK)]
            y1 = y1r[0:1, pl.ds(base, BK)]
            x2 = x2r[0:1, pl.ds(base, BK)]
            y2 = y2r[0:1, pl.ds(base, BK)]
            ar = arr[0:1, pl.ds(base, BK)]

            def run_pulls(sup_ref):
                sup_ref[...] = jnp.zeros((1, BK), jnp.float32)

                def pull(bj, carry2):
                    @pl.when(bj < bi)
                    def _inner():
                        pb = pl.multiple_of(bj * BK, BK)
                        px1 = colbuf[pl.ds(pb, BK), 0:1]
                        py1 = colbuf[pl.ds(pb, BK), 1:2]
                        px2 = colbuf[pl.ds(pb, BK), 2:3]
                        py2 = colbuf[pl.ds(pb, BK), 3:4]
                        par = colbuf[pl.ds(pb, BK), 4:5]
                        pk = colbuf[pl.ds(pb, BK), 5:6]
                        w = jnp.maximum(
                            jnp.minimum(px2, x2) - jnp.maximum(px1, x1), 0.0)
                        h = jnp.maximum(
                            jnp.minimum(py2, y2) - jnp.maximum(py1, y1), 0.0)
                        inter = w * h
                        iou = inter / (par + ar - inter + 1e-9)
                        hit = jnp.where((iou > IOU_THR) & (pk > 0.5), 1.0, 0.0)
                        sup_ref[...] = jnp.maximum(
                            sup_ref[...], jnp.max(hit, axis=0, keepdims=True))
                    return carry2

                lax.fori_loop(0, m_blocks, pull, 0)
                sup = sup_ref[...]
                gidx = base + col_i
                init = jnp.where((sup < 0.5) & (gidx < N_TOPK), 1.0, 0.0)

                tocol = lambda v: jnp.sum(
                    jnp.where(eye, v, 0.0), axis=1, keepdims=True)
                cx1, cy1 = tocol(x1), tocol(y1)
                cx2, cy2 = tocol(x2), tocol(y2)
                car = tocol(ar)
                colbuf[pl.ds(base, BK), 0:1] = cx1
                colbuf[pl.ds(base, BK), 1:2] = cy1
                colbuf[pl.ds(base, BK), 2:3] = cx2
                colbuf[pl.ds(base, BK), 3:4] = cy2
                colbuf[pl.ds(base, BK), 4:5] = car

                w_m = jnp.maximum(
                    jnp.minimum(cx2, x2) - jnp.maximum(cx1, x1), 0.0)
                h_m = jnp.maximum(
                    jnp.minimum(cy2, y2) - jnp.maximum(cy1, y1), 0.0)
                inter = w_m * h_m
                iou = inter / (car + ar - inter + 1e-9)
                amat = (iou > IOU_THR) & tri

                sup_ref[...] = init
                cnt[1] = 0

                def fix(t, c):
                    @pl.when(cnt[1] == 0)
                    def _inner():
                        k = sup_ref[...]
                        kcol = jnp.sum(
                            jnp.where(eye, k, 0.0), axis=1, keepdims=True)
                        s = jnp.max(
                            jnp.where(amat & (kcol > 0.5), 1.0, 0.0),
                            axis=0, keepdims=True)
                        knew = jnp.where(s < 0.5, init, 0.0)
                        sup_ref[...] = knew
                        cnt[1] = jnp.where(
                            jnp.sum(jnp.abs(knew - k)) == 0.0, 1, 0)
                    return c

                lax.fori_loop(0, BK, fix, 0)
                kf = sup_ref[...]
                keep_out[0:1, pl.ds(base, BK)] = kf
                colbuf[pl.ds(base, BK), 5:6] = tocol(kf)
                cnt[0] = cnt[0] + jnp.sum(kf).astype(jnp.int32)

            pl.run_scoped(run_pulls, pltpu.VMEM((1, BK), jnp.float32))
        return carry

    lax.fori_loop(0, m_blocks, block_step, 0)


def _nms_keep(x1, y1, x2, y2, ar):
    bs, w_tot = x1.shape
    spec = pl.BlockSpec((1, 1, w_tot), lambda b: (b, 0, 0))
    r3 = lambda v: v.reshape(bs, 1, w_tot)
    out = pl.pallas_call(
        _nms_body,
        grid=(bs,),
        in_specs=[spec] * 5,
        out_specs=spec,
        out_shape=jax.ShapeDtypeStruct((bs, 1, w_tot), jnp.float32),
        scratch_shapes=[
            pltpu.VMEM((w_tot, 6), jnp.float32),
            pltpu.SMEM((2,), jnp.int32),
        ],
    )(r3(x1), r3(y1), r3(x2), r3(y2), r3(ar))
    return out.reshape(bs, w_tot)


def _thresh_body(keys_ref, qmax_ref, mc_ref):
    keys = keys_ref[0]
    nq, nc = keys.shape
    qmax = qmax_ref[0]

    def step_t(it, lohi):
        lo, hi = lohi
        mid = (lo + hi) // 2
        cnt = jnp.sum(jnp.where(keys >= mid, 1, 0))
        big = cnt >= N_TOPK
        return (jnp.where(big, mid, lo), jnp.where(big, hi, mid))

    lo, hi = lax.fori_loop(
        0, 31, step_t, (jnp.int32(0), jnp.int32(0x7F800000)))
    t_val = lo
    cnt_gt = jnp.sum(jnp.where(keys > t_val, 1, 0))
    m_ties = N_TOPK - cnt_gt

    idx2d = (lax.broadcasted_iota(jnp.int32, (nq, nc), 0) * nc
             + lax.broadcasted_iota(jnp.int32, (nq, nc), 1))
    ties = keys == t_val

    def step_x(it, lohi):
        lo, hi = lohi
        mid = (lo + hi) // 2
        cnt = jnp.sum(jnp.where(ties & (idx2d < mid), 1, 0))
        big = cnt >= m_ties
        return (jnp.where(big, lo, mid), jnp.where(big, mid, hi))

    xlo, xhi = lax.fori_loop(
        0, 18, step_x, (jnp.int32(0), jnp.int32(nq * nc + 1)))
    x_cut = xhi

    selected = (keys > t_val) | (ties & (idx2d < x_cut))
    qsel = jnp.max(jnp.where(selected, 1.0, 0.0), axis=1, keepdims=True)
    mc = jnp.max(jnp.where(qsel > 0.5, qmax, -jnp.inf)) + 1.0
    mc_ref[0] = jnp.broadcast_to(mc, (1, 1))


def _max_coord(keys, qmax):
    bs, nq, nc = keys.shape
    out = pl.pallas_call(
        _thresh_body,
        grid=(bs,),
        in_specs=[pl.BlockSpec((1, nq, nc), lambda b: (b, 0, 0)),
                  pl.BlockSpec((1, nq, 1), lambda b: (b, 0, 0))],
        out_specs=pl.BlockSpec((1, 1, 1), lambda b: (b, 0, 0)),
        out_shape=jax.ShapeDtypeStruct((bs, 1, 1), jnp.float32),
    )(keys, qmax)
    return out.reshape(bs)


def _gather_cands(boxes_scaled, sc, topi, nc, mc):
    bs = sc.shape[0]
    n = sc.shape[1]
    bidx = topi // nc
    lbl = topi % nc
    box = jnp.take_along_axis(
        boxes_scaled, jnp.broadcast_to(bidx[:, :, None], (bs, n, 4)), axis=1)
    offs = lbl.astype(jnp.float32) * mc[:, None]
    boff = box + offs[:, :, None]
    x1, y1, x2, y2 = (boff[..., 0], boff[..., 1], boff[..., 2], boff[..., 3])
    areas = (x2 - x1) * (y2 - y1)
    return box, lbl, x1, y1, x2, y2, areas


def _assemble(box, sc, lbl, keepb, select, n_virtual):
    bs, w_tot = keepb.shape
    idx = jnp.arange(w_tot, dtype=jnp.int32)
    ranks = jnp.sort(
        jnp.where(keepb, idx[None, :], n_virtual), axis=1)[:, :K_OUT]
    valid = ranks < n_virtual
    inds = jnp.clip(ranks, 0, w_tot - 1)
    out_boxes = jnp.take_along_axis(
        box, jnp.broadcast_to(inds[:, :, None], (bs, K_OUT, 4)), axis=1)
    out_scores = jnp.take_along_axis(sc, inds, axis=1)
    out_labels = jnp.take_along_axis(lbl, inds, axis=1)
    valid = valid & (jnp.arange(K_OUT)[None, :] < select)
    return out_boxes, out_scores, out_labels, inds, valid


def kernel(pred_logits, pred_boxes, target_sizes,
           select_box_nums_for_evaluation):
    bs, nq, nc = pred_logits.shape
    n = N_TOPK

    scores3 = jax.nn.sigmoid(pred_logits)
    all_scores = scores3.reshape(bs, nq * nc)
    cx, cy, w, h = (pred_boxes[..., 0], pred_boxes[..., 1],
                    pred_boxes[..., 2], pred_boxes[..., 3])
    boxes_xyxy = jnp.stack(
        [cx - 0.5 * w, cy - 0.5 * h, cx + 0.5 * w, cy + 0.5 * h], axis=-1)
    img_h = target_sizes[:, 0].astype(jnp.float32)
    img_w = target_sizes[:, 1].astype(jnp.float32)
    scale_fct = jnp.stack([img_w, img_h, img_w, img_h], axis=1)
    boxes_scaled = boxes_xyxy * scale_fct[:, None, :]

    keys = lax.bitcast_convert_type(scores3, jnp.int32)
    qmax = jnp.max(boxes_scaled, axis=2, keepdims=True)
    mc = _max_coord(keys, qmax)

    scp, topip = lax.top_k(all_scores, P_PRE)
    boxp, lblp, px1, py1, px2, py2, par = _gather_cands(
        boxes_scaled, scp, topip, nc, mc)
    keep_p = _nms_keep(px1, py1, px2, py2, par)
    cnt_p = jnp.sum(keep_p, axis=1)
    prefix_ok = jnp.all(cnt_p >= K_OUT)

    def prefix_path(_):
        return _assemble(boxp, scp, lblp, keep_p > 0.5,
                         select_box_nums_for_evaluation, n)

    def full_path(_):
        sc, topi = lax.top_k(all_scores, n)
        box, lbl, x1, y1, x2, y2, areas = _gather_cands(
            boxes_scaled, sc, topi, nc, mc)
        pad = ((0, 0), (0, NPAD - n))
        keep = _nms_keep(jnp.pad(x1, pad), jnp.pad(y1, pad),
                         jnp.pad(x2, pad), jnp.pad(y2, pad),
                         jnp.pad(areas, pad))
        return _assemble(box, sc, lbl, keep[:, :n] > 0.5,
                         select_box_nums_for_evaluation, n)

    return lax.cond(prefix_ok, prefix_path, full_path, 0)


# speedup vs baseline: 198.3414x; 1.3130x over previous
import jax
import jax.numpy as jnp
from jax import lax
from jax.experimental import pallas as pl
from jax.experimental.pallas import tpu as pltpu

N_TOPK = 10000
BK = 256
NPAD = 10240
K_OUT = 300
IOU_THR = 0.7
P_PRE = 512


def _nms_body(x1r_, y1r_, x2r_, y2r_, arr_, keep_out_, colbuf, cnt):
    x1r, y1r, x2r, y2r, arr, keep_out = (
        x1r_.at[0], y1r_.at[0], x2r_.at[0], y2r_.at[0], arr_.at[0],
        keep_out_.at[0])
    w_tot = x1r_.shape[2]
    m_blocks = w_tot // BK
    keep_out[...] = jnp.zeros((1, w_tot), jnp.float32)
    cnt[0] = 0
    cnt[1] = 0

    col_i = lax.broadcasted_iota(jnp.int32, (1, BK), 1)
    row_m = lax.broadcasted_iota(jnp.int32, (BK, BK), 0)
    col_m = lax.broadcasted_iota(jnp.int32, (BK, BK), 1)
    eye = row_m == col_m
    tri = row_m < col_m

    def block_step(bi, carry):
        @pl.when(cnt[0] < K_OUT)
        def _():
            base = pl.multiple_of(bi * BK, BK)
            x1 = x1r[0:1, pl.ds(base, BK)]
            y1 = y1r[0:1, pl.ds(base, BK)]
            x2 = x2r[0:1, pl.ds(base, BK)]
            y2 = y2r[0:1, pl.ds(base, BK)]
            ar = arr[0:1, pl.ds(base, BK)]

            def run_pulls(sup_ref):
                sup_ref[...] = jnp.zeros((1, BK), jnp.float32)

                def pull(bj, carry2):
                    @pl.when(bj < bi)
                    def _inner():
                        pb = pl.multiple_of(bj * BK, BK)
                        px1 = colbuf[pl.ds(pb, BK), 0:1]
                        py1 = colbuf[pl.ds(pb, BK), 1:2]
                        px2 = colbuf[pl.ds(pb, BK), 2:3]
                        py2 = colbuf[pl.ds(pb, BK), 3:4]
                        par = colbuf[pl.ds(pb, BK), 4:5]
                        pk = colbuf[pl.ds(pb, BK), 5:6]
                        w = jnp.maximum(
                            jnp.minimum(px2, x2) - jnp.maximum(px1, x1), 0.0)
                        h = jnp.maximum(
                            jnp.minimum(py2, y2) - jnp.maximum(py1, y1), 0.0)
                        inter = w * h
                        iou = inter / (par + ar - inter + 1e-9)
                        hit = jnp.where((iou > IOU_THR) & (pk > 0.5), 1.0, 0.0)
                        sup_ref[...] = jnp.maximum(
                            sup_ref[...], jnp.max(hit, axis=0, keepdims=True))
                    return carry2

                lax.fori_loop(0, m_blocks, pull, 0)
                sup = sup_ref[...]
                gidx = base + col_i
                init = jnp.where((sup < 0.5) & (gidx < N_TOPK), 1.0, 0.0)

                tocol = lambda v: jnp.sum(
                    jnp.where(eye, v, 0.0), axis=1, keepdims=True)
                cx1, cy1 = tocol(x1), tocol(y1)
                cx2, cy2 = tocol(x2), tocol(y2)
                car = tocol(ar)
                colbuf[pl.ds(base, BK), 0:1] = cx1
                colbuf[pl.ds(base, BK), 1:2] = cy1
                colbuf[pl.ds(base, BK), 2:3] = cx2
                colbuf[pl.ds(base, BK), 3:4] = cy2
                colbuf[pl.ds(base, BK), 4:5] = car

                w_m = jnp.maximum(
                    jnp.minimum(cx2, x2) - jnp.maximum(cx1, x1), 0.0)
                h_m = jnp.maximum(
                    jnp.minimum(cy2, y2) - jnp.maximum(cy1, y1), 0.0)
                inter = w_m * h_m
                iou = inter / (car + ar - inter + 1e-9)
                amat = (iou > IOU_THR) & tri

                sup_ref[...] = init
                cnt[1] = 0

                def fix(t, c):
                    @pl.when(cnt[1] == 0)
                    def _inner():
                        k = sup_ref[...]
                        kcol = jnp.sum(
                            jnp.where(eye, k, 0.0), axis=1, keepdims=True)
                        s = jnp.max(
                            jnp.where(amat & (kcol > 0.5), 1.0, 0.0),
                            axis=0, keepdims=True)
                        knew = jnp.where(s < 0.5, init, 0.0)
                        sup_ref[...] = knew
                        cnt[1] = jnp.where(
                            jnp.sum(jnp.abs(knew - k)) == 0.0, 1, 0)
                    return c

                lax.fori_loop(0, BK, fix, 0)
                kf = sup_ref[...]
                keep_out[0:1, pl.ds(base, BK)] = kf
                colbuf[pl.ds(base, BK), 5:6] = tocol(kf)
                cnt[0] = cnt[0] + jnp.sum(kf).astype(jnp.int32)

            pl.run_scoped(run_pulls, pltpu.VMEM((1, BK), jnp.float32))
        return carry

    lax.fori_loop(0, m_blocks, block_step, 0)


def _nms_keep(x1, y1, x2, y2, ar):
    bs, w_tot = x1.shape
    spec = pl.BlockSpec((1, 1, w_tot), lambda b: (b, 0, 0))
    r3 = lambda v: v.reshape(bs, 1, w_tot)
    out = pl.pallas_call(
        _nms_body,
        grid=(bs,),
        in_specs=[spec] * 5,
        out_specs=spec,
        out_shape=jax.ShapeDtypeStruct((bs, 1, w_tot), jnp.float32),
        scratch_shapes=[
            pltpu.VMEM((w_tot, 6), jnp.float32),
            pltpu.SMEM((2,), jnp.int32),
        ],
    )(r3(x1), r3(y1), r3(x2), r3(y2), r3(ar))
    return out.reshape(bs, w_tot)


def _thresh_body(keys_ref, qmax_ref, mc_ref):
    keys = keys_ref[0]
    nq, nc = keys.shape
    qmax = qmax_ref[0]

    def step_t(it, lohi):
        lo, hi = lohi
        mid = (lo + hi) // 2
        cnt = jnp.sum(jnp.where(keys >= mid, 1, 0))
        big = cnt >= N_TOPK
        return (jnp.where(big, mid, lo), jnp.where(big, hi, mid))

    lo, hi = lax.fori_loop(
        0, 31, step_t, (jnp.int32(0), jnp.int32(0x7F800000)))
    t_val = lo
    cnt_gt = jnp.sum(jnp.where(keys > t_val, 1, 0))
    m_ties = N_TOPK - cnt_gt

    idx2d = (lax.broadcasted_iota(jnp.int32, (nq, nc), 0) * nc
             + lax.broadcasted_iota(jnp.int32, (nq, nc), 1))
    ties = keys == t_val

    def step_x(it, lohi):
        lo, hi = lohi
        mid = (lo + hi) // 2
        cnt = jnp.sum(jnp.where(ties & (idx2d < mid), 1, 0))
        big = cnt >= m_ties
        return (jnp.where(big, lo, mid), jnp.where(big, mid, hi))

    xlo, xhi = lax.fori_loop(
        0, 18, step_x, (jnp.int32(0), jnp.int32(nq * nc + 1)))
    x_cut = xhi

    selected = (keys > t_val) | (ties & (idx2d < x_cut))
    qsel = jnp.max(jnp.where(selected, 1.0, 0.0), axis=1, keepdims=True)
    mc = jnp.max(jnp.where(qsel > 0.5, qmax, -jnp.inf)) + 1.0
    mc_ref[0] = jnp.broadcast_to(mc, (1, 1))


def _max_coord(keys, qmax):
    bs, nq, nc = keys.shape
    out = pl.pallas_call(
        _thresh_body,
        grid=(bs,),
        in_specs=[pl.BlockSpec((1, nq, nc), lambda b: (b, 0, 0)),
                  pl.BlockSpec((1, nq, 1), lambda b: (b, 0, 0))],
        out_specs=pl.BlockSpec((1, 1, 1), lambda b: (b, 0, 0)),
        out_shape=jax.ShapeDtypeStruct((bs, 1, 1), jnp.float32),
    )(keys, qmax)
    return out.reshape(bs)


def _gather_cands(boxes_scaled, sc, topi, nc, mc):
    bs = sc.shape[0]
    n = sc.shape[1]
    bidx = topi // nc
    lbl = topi % nc
    box = jnp.take_along_axis(
        boxes_scaled, jnp.broadcast_to(bidx[:, :, None], (bs, n, 4)), axis=1)
    offs = lbl.astype(jnp.float32) * mc[:, None]
    boff = box + offs[:, :, None]
    x1, y1, x2, y2 = (boff[..., 0], boff[..., 1], boff[..., 2], boff[..., 3])
    areas = (x2 - x1) * (y2 - y1)
    return box, lbl, x1, y1, x2, y2, areas


def _assemble(box, sc, lbl, keepb, select, n_virtual):
    bs, w_tot = keepb.shape
    idx = jnp.arange(w_tot, dtype=jnp.int32)
    ranks = jnp.sort(
        jnp.where(keepb, idx[None, :], n_virtual), axis=1)[:, :K_OUT]
    valid = ranks < n_virtual
    inds = jnp.clip(ranks, 0, w_tot - 1)
    out_boxes = jnp.take_along_axis(
        box, jnp.broadcast_to(inds[:, :, None], (bs, K_OUT, 4)), axis=1)
    out_scores = jnp.take_along_axis(sc, inds, axis=1)
    out_labels = jnp.take_along_axis(lbl, inds, axis=1)
    valid = valid & (jnp.arange(K_OUT)[None, :] < select)
    return out_boxes, out_scores, out_labels, inds, valid


def kernel(pred_logits, pred_boxes, target_sizes,
           select_box_nums_for_evaluation):
    bs, nq, nc = pred_logits.shape
    n = N_TOPK

    scores3 = jax.nn.sigmoid(pred_logits)
    all_scores = scores3.reshape(bs, nq * nc)
    cx, cy, w, h = (pred_boxes[..., 0], pred_boxes[..., 1],
                    pred_boxes[..., 2], pred_boxes[..., 3])
    boxes_xyxy = jnp.stack(
        [cx - 0.5 * w, cy - 0.5 * h, cx + 0.5 * w, cy + 0.5 * h], axis=-1)
    img_h = target_sizes[:, 0].astype(jnp.float32)
    img_w = target_sizes[:, 1].astype(jnp.float32)
    scale_fct = jnp.stack([img_w, img_h, img_w, img_h], axis=1)
    boxes_scaled = boxes_xyxy * scale_fct[:, None, :]

    keys = lax.bitcast_convert_type(scores3, jnp.int32)
    qmax = jnp.max(boxes_scaled, axis=2, keepdims=True)
    mc = _max_coord(keys, qmax)

    nchunk = 8
    csz = (nq * nc) // nchunk
    s1, i1 = lax.top_k(all_scores.reshape(bs * nchunk, csz), P_PRE)
    off = (jnp.arange(bs * nchunk, dtype=jnp.int32) % nchunk * csz)[:, None]
    g1 = (i1 + off).reshape(bs, nchunk * P_PRE)
    s2 = s1.reshape(bs, nchunk * P_PRE)
    scp, j2 = lax.top_k(s2, P_PRE)
    topip = jnp.take_along_axis(g1, j2, axis=1)
    boxp, lblp, px1, py1, px2, py2, par = _gather_cands(
        boxes_scaled, scp, topip, nc, mc)
    keep_p = _nms_keep(px1, py1, px2, py2, par)
    cnt_p = jnp.sum(keep_p, axis=1)
    prefix_ok = jnp.all(cnt_p >= K_OUT)

    def prefix_path(_):
        return _assemble(boxp, scp, lblp, keep_p > 0.5,
                         select_box_nums_for_evaluation, n)

    def full_path(_):
        sc, topi = lax.top_k(all_scores, n)
        box, lbl, x1, y1, x2, y2, areas = _gather_cands(
            boxes_scaled, sc, topi, nc, mc)
        pad = ((0, 0), (0, NPAD - n))
        keep = _nms_keep(jnp.pad(x1, pad), jnp.pad(y1, pad),
                         jnp.pad(x2, pad), jnp.pad(y2, pad),
                         jnp.pad(areas, pad))
        return _assemble(box, sc, lbl, keep[:, :n] > 0.5,
                         select_box_nums_for_evaluation, n)

    return lax.cond(prefix_ok, prefix_path, full_path, 0)


# speedup vs baseline: 266.0056x; 1.3412x over previous
import jax
import jax.numpy as jnp
from jax import lax
from jax.experimental import pallas as pl
from jax.experimental.pallas import tpu as pltpu

N_TOPK = 10000
BK = 256
NPAD = 10240
K_OUT = 300
IOU_THR = 0.7
P_PRE = 512


def _nms_body(x1r_, y1r_, x2r_, y2r_, arr_, keep_out_, colbuf, cnt):
    x1r, y1r, x2r, y2r, arr, keep_out = (
        x1r_.at[0], y1r_.at[0], x2r_.at[0], y2r_.at[0], arr_.at[0],
        keep_out_.at[0])
    w_tot = x1r_.shape[2]
    m_blocks = w_tot // BK
    keep_out[...] = jnp.zeros((1, w_tot), jnp.float32)
    cnt[0] = 0
    cnt[1] = 0

    col_i = lax.broadcasted_iota(jnp.int32, (1, BK), 1)
    row_m = lax.broadcasted_iota(jnp.int32, (BK, BK), 0)
    col_m = lax.broadcasted_iota(jnp.int32, (BK, BK), 1)
    eye = row_m == col_m
    tri = row_m < col_m

    def block_step(bi, carry):
        @pl.when(cnt[0] < K_OUT)
        def _():
            base = pl.multiple_of(bi * BK, BK)
            x1 = x1r[0:1, pl.ds(base, BK)]
            y1 = y1r[0:1, pl.ds(base, BK)]
            x2 = x2r[0:1, pl.ds(base, BK)]
            y2 = y2r[0:1, pl.ds(base, BK)]
            ar = arr[0:1, pl.ds(base, BK)]

            def run_pulls(sup_ref):
                sup_ref[...] = jnp.zeros((1, BK), jnp.float32)

                def pull(bj, carry2):
                    @pl.when(bj < bi)
                    def _inner():
                        pb = pl.multiple_of(bj * BK, BK)
                        px1 = colbuf[pl.ds(pb, BK), 0:1]
                        py1 = colbuf[pl.ds(pb, BK), 1:2]
                        px2 = colbuf[pl.ds(pb, BK), 2:3]
                        py2 = colbuf[pl.ds(pb, BK), 3:4]
                        par = colbuf[pl.ds(pb, BK), 4:5]
                        pk = colbuf[pl.ds(pb, BK), 5:6]
                        w = jnp.maximum(
                            jnp.minimum(px2, x2) - jnp.maximum(px1, x1), 0.0)
                        h = jnp.maximum(
                            jnp.minimum(py2, y2) - jnp.maximum(py1, y1), 0.0)
                        inter = w * h
                        iou = inter / (par + ar - inter + 1e-9)
                        hit = jnp.where((iou > IOU_THR) & (pk > 0.5), 1.0, 0.0)
                        sup_ref[...] = jnp.maximum(
                            sup_ref[...], jnp.max(hit, axis=0, keepdims=True))
                    return carry2

                lax.fori_loop(0, m_blocks, pull, 0)
                sup = sup_ref[...]
                gidx = base + col_i
                init = jnp.where((sup < 0.5) & (gidx < N_TOPK), 1.0, 0.0)

                tocol = lambda v: jnp.sum(
                    jnp.where(eye, v, 0.0), axis=1, keepdims=True)
                cx1, cy1 = tocol(x1), tocol(y1)
                cx2, cy2 = tocol(x2), tocol(y2)
                car = tocol(ar)
                colbuf[pl.ds(base, BK), 0:1] = cx1
                colbuf[pl.ds(base, BK), 1:2] = cy1
                colbuf[pl.ds(base, BK), 2:3] = cx2
                colbuf[pl.ds(base, BK), 3:4] = cy2
                colbuf[pl.ds(base, BK), 4:5] = car

                w_m = jnp.maximum(
                    jnp.minimum(cx2, x2) - jnp.maximum(cx1, x1), 0.0)
                h_m = jnp.maximum(
                    jnp.minimum(cy2, y2) - jnp.maximum(cy1, y1), 0.0)
                inter = w_m * h_m
                iou = inter / (car + ar - inter + 1e-9)
                amat = (iou > IOU_THR) & tri

                sup_ref[...] = init
                cnt[1] = 0

                def fix(t, c):
                    @pl.when(cnt[1] == 0)
                    def _inner():
                        k = sup_ref[...]
                        kcol = jnp.sum(
                            jnp.where(eye, k, 0.0), axis=1, keepdims=True)
                        s = jnp.max(
                            jnp.where(amat & (kcol > 0.5), 1.0, 0.0),
                            axis=0, keepdims=True)
                        knew = jnp.where(s < 0.5, init, 0.0)
                        sup_ref[...] = knew
                        cnt[1] = jnp.where(
                            jnp.sum(jnp.abs(knew - k)) == 0.0, 1, 0)
                    return c

                lax.fori_loop(0, BK, fix, 0)
                kf = sup_ref[...]
                keep_out[0:1, pl.ds(base, BK)] = kf
                colbuf[pl.ds(base, BK), 5:6] = tocol(kf)
                cnt[0] = cnt[0] + jnp.sum(kf).astype(jnp.int32)

            pl.run_scoped(run_pulls, pltpu.VMEM((1, BK), jnp.float32))
        return carry

    lax.fori_loop(0, m_blocks, block_step, 0)


def _nms_keep(x1, y1, x2, y2, ar):
    bs, w_tot = x1.shape
    spec = pl.BlockSpec((1, 1, w_tot), lambda b: (b, 0, 0))
    r3 = lambda v: v.reshape(bs, 1, w_tot)
    out = pl.pallas_call(
        _nms_body,
        grid=(bs,),
        in_specs=[spec] * 5,
        out_specs=spec,
        out_shape=jax.ShapeDtypeStruct((bs, 1, w_tot), jnp.float32),
        scratch_shapes=[
            pltpu.VMEM((w_tot, 6), jnp.float32),
            pltpu.SMEM((2,), jnp.int32),
        ],
    )(r3(x1), r3(y1), r3(x2), r3(y2), r3(ar))
    return out.reshape(bs, w_tot)


def _thresh_body(keys_ref, qmax_ref, mc_ref):
    keys = keys_ref[0]
    nq, nc = keys.shape
    qmax = qmax_ref[0]

    def step_t(it, lohi):
        lo, hi = lohi
        mid = (lo + hi) // 2
        cnt = jnp.sum(jnp.where(keys >= mid, 1, 0))
        big = cnt >= N_TOPK
        return (jnp.where(big, mid, lo), jnp.where(big, hi, mid))

    lo, hi = lax.fori_loop(
        0, 31, step_t, (jnp.int32(0), jnp.int32(0x7F800000)))
    t_val = lo
    cnt_gt = jnp.sum(jnp.where(keys > t_val, 1, 0))
    m_ties = N_TOPK - cnt_gt

    idx2d = (lax.broadcasted_iota(jnp.int32, (nq, nc), 0) * nc
             + lax.broadcasted_iota(jnp.int32, (nq, nc), 1))
    ties = keys == t_val

    def step_x(it, lohi):
        lo, hi = lohi
        mid = (lo + hi) // 2
        cnt = jnp.sum(jnp.where(ties & (idx2d < mid), 1, 0))
        big = cnt >= m_ties
        return (jnp.where(big, lo, mid), jnp.where(big, mid, hi))

    xlo, xhi = lax.fori_loop(
        0, 18, step_x, (jnp.int32(0), jnp.int32(nq * nc + 1)))
    x_cut = xhi

    selected = (keys > t_val) | (ties & (idx2d < x_cut))
    qsel = jnp.max(jnp.where(selected, 1.0, 0.0), axis=1, keepdims=True)
    mc = jnp.max(jnp.where(qsel > 0.5, qmax, -jnp.inf)) + 1.0
    mc_ref[0] = jnp.broadcast_to(mc, (1, 1))


def _max_coord(keys, qmax):
    bs, nq, nc = keys.shape
    out = pl.pallas_call(
        _thresh_body,
        grid=(bs,),
        in_specs=[pl.BlockSpec((1, nq, nc), lambda b: (b, 0, 0)),
                  pl.BlockSpec((1, nq, 1), lambda b: (b, 0, 0))],
        out_specs=pl.BlockSpec((1, 1, 1), lambda b: (b, 0, 0)),
        out_shape=jax.ShapeDtypeStruct((bs, 1, 1), jnp.float32),
    )(keys, qmax)
    return out.reshape(bs)


def _gather_cands(boxes_scaled, sc, topi, nc, mc):
    bs = sc.shape[0]
    n = sc.shape[1]
    bidx = topi // nc
    lbl = topi % nc
    box = jnp.take_along_axis(
        boxes_scaled, jnp.broadcast_to(bidx[:, :, None], (bs, n, 4)), axis=1)
    offs = lbl.astype(jnp.float32) * mc[:, None]
    boff = box + offs[:, :, None]
    x1, y1, x2, y2 = (boff[..., 0], boff[..., 1], boff[..., 2], boff[..., 3])
    areas = (x2 - x1) * (y2 - y1)
    return box, lbl, x1, y1, x2, y2, areas


def _assemble(box, sc, lbl, keepb, select, n_virtual):
    bs, w_tot = keepb.shape
    idx = jnp.arange(w_tot, dtype=jnp.int32)
    ranks = jnp.sort(
        jnp.where(keepb, idx[None, :], n_virtual), axis=1)[:, :K_OUT]
    valid = ranks < n_virtual
    inds = jnp.clip(ranks, 0, w_tot - 1)
    out_boxes = jnp.take_along_axis(
        box, jnp.broadcast_to(inds[:, :, None], (bs, K_OUT, 4)), axis=1)
    out_scores = jnp.take_along_axis(sc, inds, axis=1)
    out_labels = jnp.take_along_axis(lbl, inds, axis=1)
    valid = valid & (jnp.arange(K_OUT)[None, :] < select)
    return out_boxes, out_scores, out_labels, inds, valid


def kernel(pred_logits, pred_boxes, target_sizes,
           select_box_nums_for_evaluation):
    bs, nq, nc = pred_logits.shape
    n = N_TOPK

    scores3 = jax.nn.sigmoid(pred_logits)
    all_scores = scores3.reshape(bs, nq * nc)
    cx, cy, w, h = (pred_boxes[..., 0], pred_boxes[..., 1],
                    pred_boxes[..., 2], pred_boxes[..., 3])
    boxes_xyxy = jnp.stack(
        [cx - 0.5 * w, cy - 0.5 * h, cx + 0.5 * w, cy + 0.5 * h], axis=-1)
    img_h = target_sizes[:, 0].astype(jnp.float32)
    img_w = target_sizes[:, 1].astype(jnp.float32)
    scale_fct = jnp.stack([img_w, img_h, img_w, img_h], axis=1)
    boxes_scaled = boxes_xyxy * scale_fct[:, None, :]

    keys = lax.bitcast_convert_type(scores3, jnp.int32)
    qmax = jnp.max(boxes_scaled, axis=2, keepdims=True)
    mc = _max_coord(keys, qmax)

    nchunk = 8
    kchunk = 128
    csz = (nq * nc) // nchunk
    s1, i1 = lax.top_k(all_scores.reshape(bs * nchunk, csz), kchunk)
    off = (jnp.arange(bs * nchunk, dtype=jnp.int32) % nchunk * csz)[:, None]
    g1 = (i1 + off).reshape(bs, nchunk * kchunk)
    s2 = s1.reshape(bs, nchunk * kchunk)
    scp, j2 = lax.top_k(s2, P_PRE)
    topip = jnp.take_along_axis(g1, j2, axis=1)
    chunk_last = s1.reshape(bs, nchunk, kchunk)[:, :, kchunk - 1]
    chunks_ok = jnp.all(chunk_last < scp[:, P_PRE - 1:P_PRE])
    boxp, lblp, px1, py1, px2, py2, par = _gather_cands(
        boxes_scaled, scp, topip, nc, mc)
    keep_p = _nms_keep(px1, py1, px2, py2, par)
    cnt_p = jnp.sum(keep_p, axis=1)
    prefix_ok = jnp.all(cnt_p >= K_OUT) & chunks_ok

    def prefix_path(_):
        return _assemble(boxp, scp, lblp, keep_p > 0.5,
                         select_box_nums_for_evaluation, n)

    def full_path(_):
        sc, topi = lax.top_k(all_scores, n)
        box, lbl, x1, y1, x2, y2, areas = _gather_cands(
            boxes_scaled, sc, topi, nc, mc)
        pad = ((0, 0), (0, NPAD - n))
        keep = _nms_keep(jnp.pad(x1, pad), jnp.pad(y1, pad),
                         jnp.pad(x2, pad), jnp.pad(y2, pad),
                         jnp.pad(areas, pad))
        return _assemble(box, sc, lbl, keep[:, :n] > 0.5,
                         select_box_nums_for_evaluation, n)

    return lax.cond(prefix_ok, prefix_path, full_path, 0)
